# Initial kernel scaffold; baseline (speedup 1.0000x reference)
import jax
import jax.numpy as jnp
from jax.experimental import pallas as pl


def kernel(x, edge_index, W1, b1, W2, b2):
    raise NotImplementedError("write your pallas kernel here")


# speedup vs baseline: 28.6022x; 28.6022x over previous
import functools

import jax
import jax.numpy as jnp
from jax import lax
from jax.experimental import pallas as pl
from jax.experimental.pallas import tpu as pltpu
from jax.experimental.pallas import tpu_sc as plsc

NC = 2
NS = 16
NW = NC * NS
LANES = 16
CHUNK = 128


def _sc_mesh():
    return plsc.VectorSubcoreMesh(
        core_axis_name="c", subcore_axis_name="s", num_cores=NC, num_subcores=NS
    )


_SC_PARAMS = pltpu.CompilerParams(
    needs_layout_passes=False, use_tc_tiling_on_sc=False
)


def _make_histogram(ept, rows):
    nid_chunks = rows // CHUNK

    @functools.partial(
        pl.kernel,
        out_type=jax.ShapeDtypeStruct((NC, rows, LANES), jnp.float32),
        mesh=_sc_mesh(),
        compiler_params=_SC_PARAMS,
        scratch_types=[
            pltpu.VMEM((ept,), jnp.int32),
            pltpu.VMEM((rows, LANES), jnp.float32),
            pltpu.VMEM((nid_chunks, CHUNK), jnp.int32),
            pltpu.VMEM_SHARED((rows, LANES), jnp.float32),
        ],
    )
    def hist_kernel(dst_h, zero_h, idc_h, cnt_h, dstv, hist, idcv, sh):
        cid = lax.axis_index("c")
        sid = lax.axis_index("s")
        wid = cid * NS + sid
        pltpu.sync_copy(dst_h.at[wid], dstv)
        pltpu.sync_copy(zero_h, hist)
        pltpu.sync_copy(idc_h, idcv)

        @pl.when(sid == 0)
        def _():
            pltpu.sync_copy(zero_h, sh)

        ones = jnp.full((LANES,), 1.0, jnp.float32)

        def body(i, carry):
            idx = dstv[pl.ds(i * LANES, LANES)]
            row = lax.shift_right_logical(idx, 4)
            col = lax.bitwise_and(idx, 15)
            plsc.addupdate_scatter(hist, [row, col], ones)
            return carry

        lax.fori_loop(0, ept // LANES, body, 0, unroll=4)

        plsc.subcore_barrier()
        for j in range(nid_chunks):
            pltpu.sync_copy(
                hist.at[pl.ds(j * CHUNK, CHUNK)], sh.at[idcv.at[j]], add=True
            )
        plsc.subcore_barrier()

        @pl.when(sid == 0)
        def _():
            pltpu.sync_copy(sh, cnt_h.at[cid])

    return hist_kernel


def _make_propagate(nch, npad):
    @functools.partial(
        pl.kernel,
        out_type=jax.ShapeDtypeStruct((NC, npad, LANES), jnp.float32),
        mesh=_sc_mesh(),
        compiler_params=_SC_PARAMS,
        scratch_types=[
            pltpu.VMEM((nch, CHUNK), jnp.int32),
            pltpu.VMEM((nch, CHUNK), jnp.int32),
            pltpu.VMEM((CHUNK, LANES), jnp.float32),
            pltpu.VMEM_SHARED((npad, LANES), jnp.float32),
        ],
    )
    def prop_kernel(src_h, dst_h, hs_h, acc_h, srcv, dstv, buf, acc_sh):
        cid = lax.axis_index("c")
        sid = lax.axis_index("s")
        wid = cid * NS + sid
        pltpu.sync_copy(src_h.at[wid], srcv)
        pltpu.sync_copy(dst_h.at[wid], dstv)

        @pl.when(sid == 0)
        def _():
            pltpu.sync_copy(hs_h, acc_sh)

        plsc.subcore_barrier()

        def body(j, carry):
            pltpu.sync_copy(hs_h.at[srcv.at[j]], buf)
            pltpu.sync_copy(buf, acc_sh.at[dstv.at[j]], add=True)
            return carry

        lax.fori_loop(0, nch, body, 0)
        plsc.subcore_barrier()

        @pl.when(sid == 0)
        def _():
            pltpu.sync_copy(acc_sh, acc_h.at[cid])

    return prop_kernel


_RB = 256


def _scale_kernel(x_ref, cnt_ref, w_ref, hs_ref, dis_ref):
    c = cnt_ref[...]
    deg = 1.0 + c[0, :] + c[1, :]
    dis = lax.rsqrt(deg)[:, None]
    h = jnp.dot(x_ref[...], w_ref[...], preferred_element_type=jnp.float32)
    hs_ref[...] = h * dis
    dis_ref[...] = dis


def _relu_kernel(acc_ref, hs_ref, dis_ref, b_ref, out_ref):
    a = acc_ref[...]
    dis = dis_ref[...]
    p = (a[0] + a[1] - hs_ref[...]) * dis + b_ref[...]
    out_ref[...] = jnp.maximum(p, 0.0) * dis


def _head_kernel(acc_ref, hs_ref, dis_ref, w_ref, b_ref, out_ref):
    a = acc_ref[...]
    p = (a[0] + a[1] - hs_ref[...]) * dis_ref[...]
    y = jnp.dot(p, w_ref[...], preferred_element_type=jnp.float32) + b_ref[...]
    m = jnp.max(y, axis=1, keepdims=True)
    e = jnp.exp(y - m)
    lse = jnp.log(jnp.sum(e, axis=1, keepdims=True))
    out_ref[...] = y - m - lse


def kernel(x, edge_index, W1, b1, W2, b2):
    n, d = x.shape
    e = edge_index.shape[1]
    h_dim = W1.shape[1]
    c_dim = W2.shape[1]
    assert h_dim == LANES

    npad = ((n + 16 * CHUNK - 1) // (16 * CHUNK)) * (16 * CHUNK)
    rows = npad // LANES
    nch = -(-e // (NW * CHUNK))
    ept = nch * CHUNK
    epad = ept * NW
    sink = npad - 1

    pad = jnp.full((epad - e,), sink, jnp.int32)
    srcp = jnp.concatenate([edge_index[0], pad]).reshape(NW, nch, CHUNK)
    dstp = jnp.concatenate([edge_index[1], pad]).reshape(NW, nch, CHUNK)
    dst_flat = dstp.reshape(NW, ept)
    xp = jnp.pad(x, ((0, npad - n), (0, 0)))
    zeros_rows = jnp.zeros((rows, LANES), jnp.float32)
    idc = jnp.arange(rows, dtype=jnp.int32).reshape(rows // CHUNK, CHUNK)

    hist = _make_histogram(ept, rows)
    prop = _make_propagate(nch, npad)

    cnt = hist(dst_flat, zeros_rows, idc)
    cnt2 = cnt.reshape(NC, npad)

    grid = (npad // _RB,)
    hs1, dis = pl.pallas_call(
        _scale_kernel,
        grid=grid,
        in_specs=[
            pl.BlockSpec((_RB, d), lambda i: (i, 0)),
            pl.BlockSpec((NC, _RB), lambda i: (0, i)),
            pl.BlockSpec((d, h_dim), lambda i: (0, 0)),
        ],
        out_specs=[
            pl.BlockSpec((_RB, h_dim), lambda i: (i, 0)),
            pl.BlockSpec((_RB, 1), lambda i: (i, 0)),
        ],
        out_shape=[
            jax.ShapeDtypeStruct((npad, h_dim), jnp.float32),
            jax.ShapeDtypeStruct((npad, 1), jnp.float32),
        ],
    )(xp, cnt2, W1)

    acc1 = prop(srcp, dstp, hs1)

    hs2 = pl.pallas_call(
        _relu_kernel,
        grid=grid,
        in_specs=[
            pl.BlockSpec((NC, _RB, h_dim), lambda i: (0, i, 0)),
            pl.BlockSpec((_RB, h_dim), lambda i: (i, 0)),
            pl.BlockSpec((_RB, 1), lambda i: (i, 0)),
            pl.BlockSpec((1, h_dim), lambda i: (0, 0)),
        ],
        out_specs=pl.BlockSpec((_RB, h_dim), lambda i: (i, 0)),
        out_shape=jax.ShapeDtypeStruct((npad, h_dim), jnp.float32),
    )(acc1, hs1, dis, b1.reshape(1, h_dim))

    acc2 = prop(srcp, dstp, hs2)

    out = pl.pallas_call(
        _head_kernel,
        grid=grid,
        in_specs=[
            pl.BlockSpec((NC, _RB, h_dim), lambda i: (0, i, 0)),
            pl.BlockSpec((_RB, h_dim), lambda i: (i, 0)),
            pl.BlockSpec((_RB, 1), lambda i: (i, 0)),
            pl.BlockSpec((h_dim, c_dim), lambda i: (0, 0)),
            pl.BlockSpec((1, c_dim), lambda i: (0, 0)),
        ],
        out_specs=pl.BlockSpec((_RB, c_dim), lambda i: (i, 0)),
        out_shape=jax.ShapeDtypeStruct((npad, c_dim), jnp.float32),
    )(acc2, hs2, dis, W2, b2.reshape(1, c_dim))

    return out[:n]


# speedup vs baseline: 38.3888x; 1.3422x over previous
import functools

import jax
import jax.numpy as jnp
from jax import lax
from jax.experimental import pallas as pl
from jax.experimental.pallas import tpu as pltpu
from jax.experimental.pallas import tpu_sc as plsc

NC = 2
NS = 16
NW = NC * NS
LANES = 16
CHUNK = 128


def _sc_mesh():
    return plsc.VectorSubcoreMesh(
        core_axis_name="c", subcore_axis_name="s", num_cores=NC, num_subcores=NS
    )


_SC_PARAMS = pltpu.CompilerParams(
    needs_layout_passes=False, use_tc_tiling_on_sc=False
)


def _make_histogram(ept, rows):
    nid_chunks = rows // CHUNK

    @functools.partial(
        pl.kernel,
        out_type=jax.ShapeDtypeStruct((NC, rows, LANES), jnp.float32),
        mesh=_sc_mesh(),
        compiler_params=_SC_PARAMS,
        scratch_types=[
            pltpu.VMEM((ept,), jnp.int32),
            pltpu.VMEM((rows, LANES), jnp.float32),
            pltpu.VMEM((nid_chunks, CHUNK), jnp.int32),
            pltpu.VMEM_SHARED((rows, LANES), jnp.float32),
        ],
    )
    def hist_kernel(dst_h, zero_h, idc_h, cnt_h, dstv, hist, idcv, sh):
        cid = lax.axis_index("c")
        sid = lax.axis_index("s")
        wid = cid * NS + sid
        pltpu.sync_copy(dst_h.at[wid], dstv)
        pltpu.sync_copy(zero_h, hist)
        pltpu.sync_copy(idc_h, idcv)

        @pl.when(sid == 0)
        def _():
            pltpu.sync_copy(zero_h, sh)

        ones = jnp.full((LANES,), 1.0, jnp.float32)

        def body(i, carry):
            idx = dstv[pl.ds(i * LANES, LANES)]
            row = lax.shift_right_logical(idx, 4)
            col = lax.bitwise_and(idx, 15)
            plsc.addupdate_scatter(hist, [row, col], ones)
            return carry

        lax.fori_loop(0, ept // LANES, body, 0, unroll=4)

        plsc.subcore_barrier()
        for j in range(nid_chunks):
            pltpu.sync_copy(
                hist.at[pl.ds(j * CHUNK, CHUNK)], sh.at[idcv.at[j]], add=True
            )
        plsc.subcore_barrier()

        @pl.when(sid == 0)
        def _():
            pltpu.sync_copy(sh, cnt_h.at[cid])

    return hist_kernel


_NBUF = 4
_SLAG = 2


def _make_propagate(nch, npad):
    @functools.partial(
        pl.kernel,
        out_type=jax.ShapeDtypeStruct((NC, npad, LANES), jnp.float32),
        mesh=_sc_mesh(),
        compiler_params=_SC_PARAMS,
        scratch_types=[
            pltpu.VMEM((nch, CHUNK), jnp.int32),
            pltpu.VMEM((nch, CHUNK), jnp.int32),
            pltpu.VMEM((_NBUF, CHUNK, LANES), jnp.float32),
            pltpu.SemaphoreType.DMA((_NBUF,)),
            pltpu.SemaphoreType.DMA((_NBUF,)),
            pltpu.VMEM_SHARED((npad, LANES), jnp.float32),
        ],
    )
    def prop_kernel(src_h, dst_h, hs_h, acc_h, srcv, dstv, bufs, gsem, ssem, acc_sh):
        cid = lax.axis_index("c")
        sid = lax.axis_index("s")
        wid = cid * NS + sid
        pltpu.sync_copy(src_h.at[wid], srcv)
        pltpu.sync_copy(dst_h.at[wid], dstv)

        @pl.when(sid == 0)
        def _():
            pltpu.sync_copy(hs_h, acc_sh)

        plsc.subcore_barrier()

        for c in range(min(_SLAG, nch)):
            pltpu.async_copy(hs_h.at[srcv.at[c]], bufs.at[c % _NBUF], gsem.at[c % _NBUF])

        def body(j, carry):
            b = lax.bitwise_and(j, _NBUF - 1)
            jg = j + _SLAG

            @pl.when(jg < nch)
            def _():
                bg = lax.bitwise_and(jg, _NBUF - 1)

                @pl.when(jg >= _NBUF)
                def _():
                    pltpu.make_async_copy(
                        bufs.at[bg], acc_sh.at[dstv.at[jg - _NBUF]], ssem.at[bg]
                    ).wait()

                pltpu.async_copy(hs_h.at[srcv.at[jg]], bufs.at[bg], gsem.at[bg])

            pltpu.make_async_copy(hs_h.at[srcv.at[j]], bufs.at[b], gsem.at[b]).wait()
            pltpu.async_copy(bufs.at[b], acc_sh.at[dstv.at[j]], ssem.at[b], add=True)
            return carry

        lax.fori_loop(0, nch, body, 0)
        for c in range(max(0, nch - _NBUF), nch):
            b = c % _NBUF
            pltpu.make_async_copy(
                bufs.at[b], acc_sh.at[dstv.at[c]], ssem.at[b]
            ).wait()
        plsc.subcore_barrier()

        @pl.when(sid == 0)
        def _():
            pltpu.sync_copy(acc_sh, acc_h.at[cid])

    return prop_kernel


_RB = 256


def _scale_kernel(x_ref, cnt_ref, w_ref, hs_ref, dis_ref):
    c = cnt_ref[...]
    deg = 1.0 + c[0, :] + c[1, :]
    dis = lax.rsqrt(deg)[:, None]
    h = jnp.dot(x_ref[...], w_ref[...], preferred_element_type=jnp.float32)
    hs_ref[...] = h * dis
    dis_ref[...] = dis


def _relu_kernel(acc_ref, hs_ref, dis_ref, b_ref, out_ref):
    a = acc_ref[...]
    dis = dis_ref[...]
    p = (a[0] + a[1] - hs_ref[...]) * dis + b_ref[...]
    out_ref[...] = jnp.maximum(p, 0.0) * dis


def _head_kernel(acc_ref, hs_ref, dis_ref, w_ref, b_ref, out_ref):
    a = acc_ref[...]
    p = (a[0] + a[1] - hs_ref[...]) * dis_ref[...]
    y = jnp.dot(p, w_ref[...], preferred_element_type=jnp.float32) + b_ref[...]
    m = jnp.max(y, axis=1, keepdims=True)
    e = jnp.exp(y - m)
    lse = jnp.log(jnp.sum(e, axis=1, keepdims=True))
    out_ref[...] = y - m - lse


def kernel(x, edge_index, W1, b1, W2, b2):
    n, d = x.shape
    e = edge_index.shape[1]
    h_dim = W1.shape[1]
    c_dim = W2.shape[1]
    assert h_dim == LANES

    npad = ((n + 16 * CHUNK - 1) // (16 * CHUNK)) * (16 * CHUNK)
    rows = npad // LANES
    nch = -(-e // (NW * CHUNK))
    ept = nch * CHUNK
    epad = ept * NW
    sink = npad - 1

    pad = jnp.full((epad - e,), sink, jnp.int32)
    srcp = jnp.concatenate([edge_index[0], pad]).reshape(NW, nch, CHUNK)
    dstp = jnp.concatenate([edge_index[1], pad]).reshape(NW, nch, CHUNK)
    dst_flat = dstp.reshape(NW, ept)
    xp = jnp.pad(x, ((0, npad - n), (0, 0)))
    zeros_rows = jnp.zeros((rows, LANES), jnp.float32)
    idc = jnp.arange(rows, dtype=jnp.int32).reshape(rows // CHUNK, CHUNK)

    hist = _make_histogram(ept, rows)
    prop = _make_propagate(nch, npad)

    cnt = hist(dst_flat, zeros_rows, idc)
    cnt2 = cnt.reshape(NC, npad)

    grid = (npad // _RB,)
    hs1, dis = pl.pallas_call(
        _scale_kernel,
        grid=grid,
        in_specs=[
            pl.BlockSpec((_RB, d), lambda i: (i, 0)),
            pl.BlockSpec((NC, _RB), lambda i: (0, i)),
            pl.BlockSpec((d, h_dim), lambda i: (0, 0)),
        ],
        out_specs=[
            pl.BlockSpec((_RB, h_dim), lambda i: (i, 0)),
            pl.BlockSpec((_RB, 1), lambda i: (i, 0)),
        ],
        out_shape=[
            jax.ShapeDtypeStruct((npad, h_dim), jnp.float32),
            jax.ShapeDtypeStruct((npad, 1), jnp.float32),
        ],
    )(xp, cnt2, W1)

    acc1 = prop(srcp, dstp, hs1)

    hs2 = pl.pallas_call(
        _relu_kernel,
        grid=grid,
        in_specs=[
            pl.BlockSpec((NC, _RB, h_dim), lambda i: (0, i, 0)),
            pl.BlockSpec((_RB, h_dim), lambda i: (i, 0)),
            pl.BlockSpec((_RB, 1), lambda i: (i, 0)),
            pl.BlockSpec((1, h_dim), lambda i: (0, 0)),
        ],
        out_specs=pl.BlockSpec((_RB, h_dim), lambda i: (i, 0)),
        out_shape=jax.ShapeDtypeStruct((npad, h_dim), jnp.float32),
    )(acc1, hs1, dis, b1.reshape(1, h_dim))

    acc2 = prop(srcp, dstp, hs2)

    out = pl.pallas_call(
        _head_kernel,
        grid=grid,
        in_specs=[
            pl.BlockSpec((NC, _RB, h_dim), lambda i: (0, i, 0)),
            pl.BlockSpec((_RB, h_dim), lambda i: (i, 0)),
            pl.BlockSpec((_RB, 1), lambda i: (i, 0)),
            pl.BlockSpec((h_dim, c_dim), lambda i: (0, 0)),
            pl.BlockSpec((1, c_dim), lambda i: (0, 0)),
        ],
        out_specs=pl.BlockSpec((_RB, c_dim), lambda i: (i, 0)),
        out_shape=jax.ShapeDtypeStruct((npad, c_dim), jnp.float32),
    )(acc2, hs2, dis, W2, b2.reshape(1, c_dim))

    return out[:n]
